# Initial kernel scaffold; baseline (speedup 1.0000x reference)
import jax
import jax.numpy as jnp
from jax.experimental import pallas as pl


def kernel(coordinates, nuclear_charges, natom_counts):
    raise NotImplementedError("write your pallas kernel here")


# speedup vs baseline: 17.5387x; 17.5387x over previous
import math

import jax
import jax.numpy as jnp
import numpy as np
from jax.experimental import pallas as pl
from jax.experimental.pallas import tpu as pltpu

_SPECIES = (1, 6, 7, 8)
_HIGH_CUTOFF = 6.0
_N_GAUSS = 20
_W = 2.0
_LMAX = 2
_B, _N = 16, 96

_OFFSETS = np.linspace(0.0, _HIGH_CUTOFF, _N_GAUSS + 1, dtype=np.float32)[1:]
_SQRTPI = float(np.sqrt(np.pi))
_PI = float(np.pi)

_ANG_BY_L = {
    0: [((0, 0, 0), 1.0)],
    1: [((1, 0, 0), 1.0), ((0, 1, 0), 1.0), ((0, 0, 1), 1.0)],
    2: [((2, 0, 0), 1.0), ((1, 1, 0), 2.0), ((0, 2, 0), 1.0),
        ((1, 0, 1), 2.0), ((0, 1, 1), 2.0), ((0, 0, 2), 1.0)],
}
_COMBOS = ((0, 1), (0, 2), (0, 3), (1, 2), (1, 3), (2, 3))


def _fp_kernel(xc_ref, xr_ref, z_ref, cnt_ref, out_ref):
    f32 = jnp.float32
    xc = xc_ref[0]
    xr = xr_ref[0]
    z = z_ref[0]
    natom = cnt_ref[0, 0, 0]

    n = _N
    dx = xr[0:1, :] - xc[:, 0:1]
    dy = xr[1:2, :] - xc[:, 1:2]
    dz = xr[2:3, :] - xc[:, 2:3]

    d2 = jnp.maximum(dx * dx + dy * dy + dz * dz, 1e-12)
    dist = jnp.sqrt(d2)
    jj = jax.lax.broadcasted_iota(jnp.int32, (n, n), 0)
    ii = jax.lax.broadcasted_iota(jnp.int32, (n, n), 1)
    valid = (dist < _HIGH_CUTOFF) & (ii != jj) & (jj < natom)
    coeffs = valid.astype(f32)
    dist_s = jnp.where(valid, dist, 1.0)
    dxs = jnp.where(valid, dx, 1.0)
    dys = jnp.where(valid, dy, 1.0)
    dzs = jnp.where(valid, dz, 1.0)

    inv_d = 1.0 / dist_s
    inv_d2 = inv_d * inv_d
    cut = 0.5 * (jnp.cos(dist_s * (_PI / _HIGH_CUTOFF)) + 1.0)
    sigma2 = jnp.log(1.0 + _W * inv_d2)
    mu = jnp.log(dist_s) - 0.5 * sigma2
    rsig = jax.lax.rsqrt(sigma2)
    neg_half_inv_s2 = -0.5 / sigma2
    base = cut * coeffs * rsig

    rad = []
    for g in range(_N_GAUSS):
        c = 1.0 / (float(_OFFSETS[g]) * _SQRTPI)
        cen = float(np.log(_OFFSETS[g])) - mu
        rad.append((c * base) * jnp.exp(cen * cen * neg_half_inv_s2))

    u2 = inv_d2 * coeffs
    u3 = u2 * inv_d
    u4 = u2 * inv_d2
    mono = {(0, 0, 0): 1.0,
            (1, 0, 0): dxs, (0, 1, 0): dys, (0, 0, 1): dzs,
            (2, 0, 0): dxs * dxs, (1, 1, 0): dxs * dys, (0, 2, 0): dys * dys,
            (1, 0, 1): dxs * dzs, (0, 1, 1): dys * dzs, (0, 0, 2): dzs * dzs}
    ubyl = {0: u2, 1: u3, 2: u4}
    ang_by_l = {}
    for l, comps in _ANG_BY_L.items():
        lst = []
        for (nmk, wt) in comps:
            m = mono[nmk]
            sw = math.sqrt(wt)
            if isinstance(m, float):
                lst.append(ubyl[l] * (sw * m))
            else:
                lst.append((ubyl[l] * sw) * m if sw != 1.0 else ubyl[l] * m)
        ang_by_l[l] = lst

    m4 = jnp.concatenate(
        [(z == s).astype(f32) for s in _SPECIES], axis=0)
    lane_i = jax.lax.broadcasted_iota(jnp.int32, (1, n), 1)
    valid_i = (lane_i < natom).astype(f32)

    for l in range(_LMAX + 1):
        for g in range(_N_GAUSS):
            acc = None
            for a_arr in ang_by_l[l]:
                p = a_arr * rad[g]
                t = jax.lax.dot(m4, p, preferred_element_type=f32)
                t0, t1, t2, t3 = t[0:1], t[1:2], t[2:3], t[3:4]
                o = jnp.concatenate(
                    [t0 * t0, t1 * t1, t2 * t2, t3 * t3,
                     2.0 * (t0 * t1), 2.0 * (t0 * t2), 2.0 * (t0 * t3),
                     2.0 * (t1 * t2), 2.0 * (t1 * t3), 2.0 * (t2 * t3)],
                    axis=0)
                acc = o if acc is None else acc + o
            out_ref[0, l * _N_GAUSS + g] = acc * valid_i


def kernel(coordinates, nuclear_charges, natom_counts):
    b, n, _ = coordinates.shape
    xc = coordinates.astype(jnp.float32)
    xr = jnp.transpose(xc, (0, 2, 1))
    z = nuclear_charges.astype(jnp.int32).reshape(b, 1, n)
    cnt = natom_counts.astype(jnp.int32).reshape(b, 1, 1)

    out = pl.pallas_call(
        _fp_kernel,
        grid=(b,),
        in_specs=[
            pl.BlockSpec((1, n, 3), lambda i: (i, 0, 0)),
            pl.BlockSpec((1, 3, n), lambda i: (i, 0, 0)),
            pl.BlockSpec((1, 1, n), lambda i: (i, 0, 0)),
            pl.BlockSpec((1, 1, 1), lambda i: (i, 0, 0)),
        ],
        out_specs=pl.BlockSpec((1, 60, 10, n), lambda i: (i, 0, 0, 0)),
        out_shape=jax.ShapeDtypeStruct((b, 60, 10, n), jnp.float32),
        compiler_params=pltpu.CompilerParams(
            dimension_semantics=("parallel",)),
    )(xc, xr, z, cnt)

    fp = out.reshape(b, 3, _N_GAUSS, 10, n)
    fp = jnp.transpose(fp, (0, 4, 1, 3, 2))
    return fp.reshape(b, n, 3 * 10 * _N_GAUSS)
